# baseline (speedup 1.0000x reference)
import functools

import jax
import jax.numpy as jnp
from jax import lax
from jax.experimental import pallas as pl
from jax.experimental.pallas import tpu as pltpu
from jax.experimental.pallas import tpu_sc as plsc

_B, _L, _I, _H, _C, _E = 4, 2048, 1024, 2048, 8192, 256
_M = _B * _L
_MT = 1024
_CT = 1024
_NM = _M // _MT
_NC = _C // _CT


def _mlp_body(x_ref, wh_ref, bh_ref, wl_ref, bl_ref,
              logits_ref, idx_ref, h_ref, vmax_ref, vidx_ref):
    c = pl.program_id(1)

    @pl.when(c == 0)
    def _():
        h = jnp.dot(x_ref[...], wh_ref[...], preferred_element_type=jnp.float32, precision=lax.Precision.DEFAULT)
        h_ref[...] = jnp.maximum(h + bh_ref[...], 0.0)
        vmax_ref[...] = jnp.full((_MT, 1), -jnp.inf, jnp.float32)
        vidx_ref[...] = jnp.zeros((_MT, 1), jnp.int32)

    _W = _CT // 2
    for off in (0, _W):
        sl = pl.ds(off, _W)
        tile = jnp.dot(h_ref[...], wl_ref[:, sl],
                       preferred_element_type=jnp.float32,
                       precision=lax.Precision.DEFAULT)
        tile = tile + bl_ref[:, sl]
        logits_ref[:, sl] = tile

        tmax = jnp.max(tile, axis=1, keepdims=True)
        iota = lax.broadcasted_iota(jnp.int32, tile.shape, 1)
        tix = jnp.min(jnp.where(tile == tmax, iota, jnp.int32(2**30)),
                      axis=1, keepdims=True) + c * _CT + off
        better = tmax > vmax_ref[...]
        vidx_ref[...] = jnp.where(better, tix, vidx_ref[...])
        vmax_ref[...] = jnp.maximum(tmax, vmax_ref[...])

    @pl.when(c == _NC - 1)
    def _():
        vt = vidx_ref[...].T.reshape(1, 1, _MT)
        idx_ref[...] = jnp.broadcast_to(vt, (1, 8, _MT))


def _mlp_logits_argmax(x, W_h, b_h, W_l, b_l):
    return pl.pallas_call(
        _mlp_body,
        grid=(_NM, _NC),
        in_specs=[
            pl.BlockSpec((_MT, _I), lambda m, c: (m, 0)),
            pl.BlockSpec((_I, _H), lambda m, c: (0, 0)),
            pl.BlockSpec((1, _H), lambda m, c: (0, 0)),
            pl.BlockSpec((_H, _CT), lambda m, c: (0, c)),
            pl.BlockSpec((1, _CT), lambda m, c: (0, c)),
        ],
        out_specs=[
            pl.BlockSpec((_MT, _CT), lambda m, c: (m, c)),
            pl.BlockSpec((1, 8, _MT), lambda m, c: (m, 0, 0)),
        ],
        out_shape=[
            jax.ShapeDtypeStruct((_M, _C), jnp.float32),
            jax.ShapeDtypeStruct((_NM, 8, _MT), jnp.int32),
        ],
        scratch_shapes=[
            pltpu.VMEM((_MT, _H), jnp.float32),
            pltpu.VMEM((_MT, 1), jnp.float32),
            pltpu.VMEM((_MT, 1), jnp.int32),
        ],
        compiler_params=pltpu.CompilerParams(
            dimension_semantics=("arbitrary", "arbitrary"),
        ),
    )(x, W_h, b_h, W_l, b_l)


def _gather_codes(table, idx):
    info = plsc.get_sparse_core_info()
    nw = info.num_cores * info.num_subcores
    b_per_w = _M // nw
    mesh = plsc.VectorSubcoreMesh(core_axis_name="c", subcore_axis_name="s")

    half = b_per_w // 2

    @functools.partial(
        pl.kernel,
        out_type=jax.ShapeDtypeStruct((_M, _E), jnp.float32),
        mesh=mesh,
        scratch_types=[
            pltpu.VMEM((b_per_w,), jnp.int32),
            pltpu.VMEM((half, _E), jnp.float32),
            pltpu.VMEM((half, _E), jnp.float32),
            pltpu.SemaphoreType.DMA,
            pltpu.SemaphoreType.DMA,
            pltpu.SemaphoreType.DMA,
        ],
    )
    def k(table_hbm, idx_hbm, out_hbm, idx_v, rows0_v, rows1_v,
          gsem0, gsem1, ssem):
        wid = lax.axis_index("s") * info.num_cores + lax.axis_index("c")
        base = wid * b_per_w
        pltpu.sync_copy(idx_hbm.at[pl.ds(base, b_per_w)], idx_v)
        g0 = pltpu.async_copy(table_hbm.at[idx_v.at[pl.ds(0, half)]],
                              rows0_v, gsem0)
        g1 = pltpu.async_copy(table_hbm.at[idx_v.at[pl.ds(half, half)]],
                              rows1_v, gsem1)
        g0.wait()
        s0 = pltpu.async_copy(rows0_v, out_hbm.at[pl.ds(base, half)], ssem)
        g1.wait()
        pltpu.sync_copy(rows1_v, out_hbm.at[pl.ds(base + half, half)])
        s0.wait()

    return k(table, idx)


def kernel(inputs_BxLxI, W_h, b_h, W_l, b_l, W_cb, testing):
    x = inputs_BxLxI.reshape(_M, _I)
    logits, idx3 = _mlp_logits_argmax(x, W_h, b_h.reshape(1, _H),
                                      W_l, b_l.reshape(1, _C))
    idx = idx3[:, 0, :].reshape(_M)
    codes = _gather_codes(W_cb, idx)
    return logits.reshape(_B, _L, _C), codes.reshape(_B, _L, _E)


# speedup vs baseline: 1.0028x; 1.0028x over previous
import functools

import jax
import jax.numpy as jnp
from jax import lax
from jax.experimental import pallas as pl
from jax.experimental.pallas import tpu as pltpu
from jax.experimental.pallas import tpu_sc as plsc

_B, _L, _I, _H, _C, _E = 4, 2048, 1024, 2048, 8192, 256
_M = _B * _L
_MT = 1024
_CT = 1024
_NM = _M // _MT
_NC = _C // _CT


def _mlp_body(x_ref, wh_ref, bh_ref, wl_ref, bl_ref,
              logits_ref, idx_ref, h_ref, vmax_ref, vidx_ref):
    c = pl.program_id(1)

    @pl.when(c == 0)
    def _():
        h = jnp.dot(x_ref[...], wh_ref[...], preferred_element_type=jnp.float32, precision=lax.Precision.DEFAULT)
        h_ref[...] = jnp.maximum(h + bh_ref[...], 0.0)
        vmax_ref[...] = jnp.full((_MT, 1), -jnp.inf, jnp.float32)
        vidx_ref[...] = jnp.zeros((_MT, 1), jnp.int32)

    _W = _CT // 2
    for off in (0, _W):
        sl = pl.ds(off, _W)
        tile = jnp.dot(h_ref[...], wl_ref[:, sl],
                       preferred_element_type=jnp.float32,
                       precision=lax.Precision.DEFAULT)
        tile = tile + bl_ref[:, sl]
        logits_ref[:, sl] = tile

        tmax = jnp.max(tile, axis=1, keepdims=True)
        iota = lax.broadcasted_iota(jnp.int32, tile.shape, 1)
        tix = jnp.min(jnp.where(tile == tmax, iota, jnp.int32(2**30)),
                      axis=1, keepdims=True) + c * _CT + off
        better = tmax > vmax_ref[...]
        vidx_ref[...] = jnp.where(better, tix, vidx_ref[...])
        vmax_ref[...] = jnp.maximum(tmax, vmax_ref[...])

    @pl.when(c == _NC - 1)
    def _():
        vt = vidx_ref[...].T.reshape(1, 1, _MT)
        idx_ref[...] = jnp.broadcast_to(vt, (1, 8, _MT))


def _mlp_logits_argmax(x, W_h, b_h, W_l, b_l):
    return pl.pallas_call(
        _mlp_body,
        grid=(_NM, _NC),
        in_specs=[
            pl.BlockSpec((_MT, _I), lambda m, c: (m, 0)),
            pl.BlockSpec((_I, _H), lambda m, c: (0, 0)),
            pl.BlockSpec((1, _H), lambda m, c: (0, 0)),
            pl.BlockSpec((_H, _CT), lambda m, c: (0, c)),
            pl.BlockSpec((1, _CT), lambda m, c: (0, c)),
        ],
        out_specs=[
            pl.BlockSpec((_MT, _CT), lambda m, c: (m, c)),
            pl.BlockSpec((1, 8, _MT), lambda m, c: (m, 0, 0)),
        ],
        out_shape=[
            jax.ShapeDtypeStruct((_M, _C), jnp.float32),
            jax.ShapeDtypeStruct((_NM, 8, _MT), jnp.int32),
        ],
        scratch_shapes=[
            pltpu.VMEM((_MT, _H), jnp.float32),
            pltpu.VMEM((_MT, 1), jnp.float32),
            pltpu.VMEM((_MT, 1), jnp.int32),
        ],
        compiler_params=pltpu.CompilerParams(
            dimension_semantics=("parallel", "arbitrary"),
        ),
    )(x, W_h, b_h, W_l, b_l)


def _gather_codes(table, idx):
    info = plsc.get_sparse_core_info()
    nw = info.num_cores * info.num_subcores
    b_per_w = _M // nw
    mesh = plsc.VectorSubcoreMesh(core_axis_name="c", subcore_axis_name="s")

    half = b_per_w // 2

    @functools.partial(
        pl.kernel,
        out_type=jax.ShapeDtypeStruct((_M, _E), jnp.float32),
        mesh=mesh,
        scratch_types=[
            pltpu.VMEM((b_per_w,), jnp.int32),
            pltpu.VMEM((half, _E), jnp.float32),
            pltpu.VMEM((half, _E), jnp.float32),
            pltpu.SemaphoreType.DMA,
            pltpu.SemaphoreType.DMA,
            pltpu.SemaphoreType.DMA,
        ],
    )
    def k(table_hbm, idx_hbm, out_hbm, idx_v, rows0_v, rows1_v,
          gsem0, gsem1, ssem):
        wid = lax.axis_index("s") * info.num_cores + lax.axis_index("c")
        base = wid * b_per_w
        pltpu.sync_copy(idx_hbm.at[pl.ds(base, b_per_w)], idx_v)
        g0 = pltpu.async_copy(table_hbm.at[idx_v.at[pl.ds(0, half)]],
                              rows0_v, gsem0)
        g1 = pltpu.async_copy(table_hbm.at[idx_v.at[pl.ds(half, half)]],
                              rows1_v, gsem1)
        g0.wait()
        s0 = pltpu.async_copy(rows0_v, out_hbm.at[pl.ds(base, half)], ssem)
        g1.wait()
        pltpu.sync_copy(rows1_v, out_hbm.at[pl.ds(base + half, half)])
        s0.wait()

    return k(table, idx)


def kernel(inputs_BxLxI, W_h, b_h, W_l, b_l, W_cb, testing):
    x = inputs_BxLxI.reshape(_M, _I)
    logits, idx3 = _mlp_logits_argmax(x, W_h, b_h.reshape(1, _H),
                                      W_l, b_l.reshape(1, _C))
    idx = idx3[:, 0, :].reshape(_M)
    codes = _gather_codes(W_cb, idx)
    return logits.reshape(_B, _L, _C), codes.reshape(_B, _L, _E)
